# baseline (speedup 1.0000x reference)
import functools

import jax
import jax.numpy as jnp
from jax import lax
from jax.experimental import pallas as pl
from jax.experimental.pallas import tpu as pltpu
from jax.experimental.pallas import tpu_sc as plsc

V = 100000
D = 64
B = 1024

VBLK = 2048
NBLK = (V + VBLK - 1) // VBLK


_NC = 2
_NS = 16
_NW = _NC * _NS
_B_PER_W = B // _NW


def _sc_gather(ids, table):
    mesh = plsc.VectorSubcoreMesh(core_axis_name="c", subcore_axis_name="s")

    @functools.partial(
        pl.kernel,
        out_type=jax.ShapeDtypeStruct((B, D), jnp.float32),
        mesh=mesh,
        scratch_types=[
            pltpu.VMEM((_B_PER_W,), jnp.int32),
            pltpu.VMEM((_B_PER_W, D), jnp.float32),
            pltpu.SemaphoreType.DMA,
        ],
        compiler_params=pltpu.CompilerParams(use_tc_tiling_on_sc=False),
    )
    def gather_kernel(ids_hbm, table_hbm, out_hbm, idx_v, rows_v, sem):
        wid = lax.axis_index("s") * _NC + lax.axis_index("c")
        base = wid * _B_PER_W
        pltpu.sync_copy(ids_hbm.at[pl.ds(base, _B_PER_W)], idx_v)
        pltpu.async_copy(table_hbm.at[idx_v], rows_v, sem).wait()
        pltpu.sync_copy(rows_v, out_hbm.at[pl.ds(base, _B_PER_W)])

    return gather_kernel(ids, table)


def _proj_kernel(x_ref, w_ref, b_ref, out_ref, loss_ref, sw_acc, sb_acc):
    i = pl.program_id(0)
    x = x_ref[...]
    w = w_ref[...]
    b = b_ref[...]

    acc = lax.dot_general(
        x.astype(jnp.bfloat16),
        w.astype(jnp.bfloat16),
        (((1,), (1,)), ((), ())),
        preferred_element_type=jnp.float32,
    )
    out_ref[...] = acc + b

    rows = lax.broadcasted_iota(jnp.int32, (VBLK, 1), 0)
    valid = (i * VBLK + rows) < V
    sw = jnp.sum(jnp.where(valid, w, 0.0), axis=0, keepdims=True)
    sb = jnp.sum(jnp.where(valid.T, b, 0.0))

    @pl.when(i == 0)
    def _():
        sw_acc[...] = jnp.zeros_like(sw_acc)
        sb_acc[0] = 0.0

    sw_acc[...] += sw
    sb_acc[0] += sb

    @pl.when(i == NBLK - 1)
    def _():
        sx = jnp.sum(x, axis=0, keepdims=True)
        total = jnp.sum(sx * sw_acc[...]) + B * sb_acc[0]
        loss_ref[...] = jnp.full((1, 1), total / (B * V), jnp.float32)


def kernel(input_ids, embed_table, proj_w, proj_b):
    x = _sc_gather(input_ids, embed_table)
    b2d = proj_b.reshape(1, V)
    logits, loss2d = pl.pallas_call(
        _proj_kernel,
        grid=(NBLK,),
        in_specs=[
            pl.BlockSpec((B, D), lambda i: (0, 0)),
            pl.BlockSpec((VBLK, D), lambda i: (i, 0)),
            pl.BlockSpec((1, VBLK), lambda i: (0, i)),
        ],
        out_specs=[
            pl.BlockSpec((B, VBLK), lambda i: (0, i)),
            pl.BlockSpec((1, 1), lambda i: (0, 0)),
        ],
        out_shape=[
            jax.ShapeDtypeStruct((B, V), jnp.float32),
            jax.ShapeDtypeStruct((1, 1), jnp.float32),
        ],
        scratch_shapes=[
            pltpu.VMEM((1, D), jnp.float32),
            pltpu.SMEM((1,), jnp.float32),
        ],
    )(x, proj_w, b2d)
    loss = loss2d[0, 0]
    return (loss, logits)


# speedup vs baseline: 1.0049x; 1.0049x over previous
import functools

import jax
import jax.numpy as jnp
from jax import lax
from jax.experimental import pallas as pl
from jax.experimental.pallas import tpu as pltpu
from jax.experimental.pallas import tpu_sc as plsc

V = 100000
D = 64
B = 1024

VBLK = 4096
NBLK = (V + VBLK - 1) // VBLK


_NC = 2
_NS = 16
_NW = _NC * _NS
_B_PER_W = B // _NW


def _sc_gather(ids, table):
    mesh = plsc.VectorSubcoreMesh(core_axis_name="c", subcore_axis_name="s")

    @functools.partial(
        pl.kernel,
        out_type=jax.ShapeDtypeStruct((B, D), jnp.float32),
        mesh=mesh,
        scratch_types=[
            pltpu.VMEM((_B_PER_W,), jnp.int32),
            pltpu.VMEM((_B_PER_W, D), jnp.float32),
            pltpu.SemaphoreType.DMA,
        ],
        compiler_params=pltpu.CompilerParams(use_tc_tiling_on_sc=False),
    )
    def gather_kernel(ids_hbm, table_hbm, out_hbm, idx_v, rows_v, sem):
        wid = lax.axis_index("s") * _NC + lax.axis_index("c")
        base = wid * _B_PER_W
        pltpu.sync_copy(ids_hbm.at[pl.ds(base, _B_PER_W)], idx_v)
        pltpu.async_copy(table_hbm.at[idx_v], rows_v, sem).wait()
        pltpu.sync_copy(rows_v, out_hbm.at[pl.ds(base, _B_PER_W)])

    return gather_kernel(ids, table)


def _proj_kernel(x_ref, w_ref, b_ref, out_ref, loss_ref, sw_acc, sb_acc):
    i = pl.program_id(0)
    x = x_ref[...]
    w = w_ref[...]
    b = b_ref[...]

    acc = lax.dot_general(
        x.astype(jnp.bfloat16),
        w.astype(jnp.bfloat16),
        (((1,), (1,)), ((), ())),
        preferred_element_type=jnp.float32,
    )
    out_ref[...] = acc + b

    rows = lax.broadcasted_iota(jnp.int32, (VBLK, 1), 0)
    valid = (i * VBLK + rows) < V
    sw = jnp.sum(jnp.where(valid, w, 0.0), axis=0, keepdims=True)
    sb = jnp.sum(jnp.where(valid.T, b, 0.0))

    @pl.when(i == 0)
    def _():
        sw_acc[...] = jnp.zeros_like(sw_acc)
        sb_acc[0] = 0.0

    sw_acc[...] += sw
    sb_acc[0] += sb

    @pl.when(i == NBLK - 1)
    def _():
        sx = jnp.sum(x, axis=0, keepdims=True)
        total = jnp.sum(sx * sw_acc[...]) + B * sb_acc[0]
        loss_ref[...] = jnp.full((1, 1), total / (B * V), jnp.float32)


def kernel(input_ids, embed_table, proj_w, proj_b):
    x = _sc_gather(input_ids, embed_table)
    b2d = proj_b.reshape(1, V)
    logits, loss2d = pl.pallas_call(
        _proj_kernel,
        grid=(NBLK,),
        in_specs=[
            pl.BlockSpec((B, D), lambda i: (0, 0)),
            pl.BlockSpec((VBLK, D), lambda i: (i, 0)),
            pl.BlockSpec((1, VBLK), lambda i: (0, i)),
        ],
        out_specs=[
            pl.BlockSpec((B, VBLK), lambda i: (0, i)),
            pl.BlockSpec((1, 1), lambda i: (0, 0)),
        ],
        out_shape=[
            jax.ShapeDtypeStruct((B, V), jnp.float32),
            jax.ShapeDtypeStruct((1, 1), jnp.float32),
        ],
        scratch_shapes=[
            pltpu.VMEM((1, D), jnp.float32),
            pltpu.SMEM((1,), jnp.float32),
        ],
    )(x, proj_w, b2d)
    loss = loss2d[0, 0]
    return (loss, logits)
